# baseline (speedup 1.0000x reference)
import functools

import jax
import jax.numpy as jnp
import numpy as np
from jax import lax
from jax.experimental import pallas as pl
from jax.experimental.pallas import tpu as pltpu
from jax.experimental.pallas import tpu_sc as plsc

_C = 128
_H = 128
_W = 128
_L = 8192
_S = 32
_F = 8
_NC = 2
_NS = 16
_PPT = 4
_PLANE = _H * _W
_LH = _L // _NC
_CHUNK = 128
_NPAIR = _LH // (2 * _CHUNK)
_GPC = _CHUNK // 16

_TVALS = [float(t) for t in np.linspace(0.0, 1.0, _S).astype(np.float32)]
_STEP = float(np.float32(1.0 / (_S - 1)))


def _body(feat_hbm, lines_hbm, out_hbm, feat_v, lines_v, stages, sems):
    lh = lax.axis_index("c")
    pp = lax.axis_index("s")

    pltpu.sync_copy(feat_hbm.at[pl.ds(pp * (_PPT * _PLANE), _PPT * _PLANE)],
                    feat_v)
    pltpu.sync_copy(lines_hbm.at[pl.ds(lh * (_LH * 4), _LH * 4)], lines_v)

    lane = lax.iota(jnp.int32, 16)
    planes = [feat_v.at[pl.ds(c * _PLANE, _PLANE)] for c in range(_PPT)]
    cols = [jnp.full((16,), k, jnp.int32) for k in range(2 * _PPT * _F)]
    interleave = plsc.PackFormat.INTERLEAVED

    def make_group_body(stage):
        def group_body(gi, lloc0):
            lloc = lloc0 + gi * 16
            li4 = (lloc + lane) * 4
            x1 = plsc.load_gather(lines_v, [li4])
            y1 = plsc.load_gather(lines_v, [li4 + 1])
            x2 = plsc.load_gather(lines_v, [li4 + 2])
            y2 = plsc.load_gather(lines_v, [li4 + 3])
            sx = (x1 - x2) * _STEP
            sy = (y1 - y2) * _STEP
            px = x2 - 0.5
            py = y2 - 0.5
            rows = gi * 16 + lane
            for f in range(_F):
                acc = [None] * _PPT
                for j in range(4):
                    x0i = px.astype(jnp.int32)
                    y0i = py.astype(jnp.int32)
                    x1i = jnp.minimum(x0i + 1, _W - 1)
                    y1i = jnp.minimum(y0i + 1, _H - 1)
                    wb = px - x0i.astype(jnp.float32)
                    wa = x1i.astype(jnp.float32) - px
                    wd = py - y0i.astype(jnp.float32)
                    wc = y1i.astype(jnp.float32) - py
                    w00 = plsc.pack(wa * wc, wa * wc, format=interleave)
                    w01 = plsc.pack(wb * wc, wb * wc, format=interleave)
                    w10 = plsc.pack(wa * wd, wa * wd, format=interleave)
                    w11 = plsc.pack(wb * wd, wb * wd, format=interleave)
                    iy0 = y0i * _W
                    iy1 = y1i * _W
                    i00 = iy0 + x0i
                    i01 = iy0 + x1i
                    i10 = iy1 + x0i
                    i11 = iy1 + x1i
                    for c in range(_PPT):
                        g00 = plsc.bitcast(
                            plsc.load_gather(planes[c], [i00]), jnp.bfloat16)
                        g01 = plsc.bitcast(
                            plsc.load_gather(planes[c], [i01]), jnp.bfloat16)
                        g10 = plsc.bitcast(
                            plsc.load_gather(planes[c], [i10]), jnp.bfloat16)
                        g11 = plsc.bitcast(
                            plsc.load_gather(planes[c], [i11]), jnp.bfloat16)
                        v = w00 * g00 + w01 * g01 + w10 * g10 + w11 * g11
                        acc[c] = v if j == 0 else jnp.maximum(acc[c], v)
                    px = px + sx
                    py = py + sy
                for c in range(_PPT):
                    vlo, vhi = plsc.unpack(acc[c], format=interleave,
                                           preferred_element_type=jnp.float32)
                    plsc.store_scatter(stage, [rows, cols[c * _F + f]], vlo)
                    plsc.store_scatter(stage, [rows, cols[32 + c * _F + f]],
                                       vhi)
            return lloc0

        return group_body

    def out_slice(ci):
        return out_hbm.at[lh, pp, pl.ds(ci * _CHUNK, _CHUNK), :]

    def pair_body(cp, carry):
        for b in range(2):
            ci = cp * 2 + b
            stage = stages[b]
            dst = out_slice(ci)

            @pl.when(cp > 0)
            def _():
                pltpu.make_async_copy(stage, dst, sems[b]).wait()

            lax.fori_loop(0, _GPC, make_group_body(stage), ci * _CHUNK)
            pltpu.async_copy(stage, dst, sems[b])
        return carry

    lax.fori_loop(0, _NPAIR, pair_body, 0)
    for b in range(2):
        pltpu.make_async_copy(stages[b], out_slice(2 * (_NPAIR - 1) + b),
                              sems[b]).wait()


_sc_call = functools.partial(
    pl.kernel,
    out_type=jax.ShapeDtypeStruct((_NC, _NS, _LH, 2 * _PPT * _F),
                                  jnp.float32),
    mesh=plsc.VectorSubcoreMesh(core_axis_name="c", subcore_axis_name="s"),
    compiler_params=pltpu.CompilerParams(use_tc_tiling_on_sc=False,
                                         needs_layout_passes=False),
    scratch_types=[
        pltpu.VMEM((_PPT * _PLANE,), jnp.int32),
        pltpu.VMEM((_LH * 4,), jnp.float32),
        [pltpu.VMEM((_CHUNK, 2 * _PPT * _F), jnp.float32)] * 2,
        [pltpu.SemaphoreType.DMA] * 2,
    ],
)(_body)


def kernel(features_per_image, lines_per_im):
    fb = features_per_image.astype(jnp.bfloat16)
    lo = lax.bitcast_convert_type(fb[:64], jnp.uint16).astype(jnp.uint32)
    hi = lax.bitcast_convert_type(fb[64:], jnp.uint16).astype(jnp.uint32)
    packed = lax.bitcast_convert_type((hi << 16) | lo, jnp.int32).reshape(-1)
    lines_flat = lines_per_im.reshape(-1)
    raw = _sc_call(packed, lines_flat)
    return (raw.reshape(_NC, _NS, _LH, 2, 32)
            .transpose(0, 2, 3, 1, 4)
            .reshape(_L, _C * _F))


# speedup vs baseline: 1.3468x; 1.3468x over previous
import functools

import jax
import jax.numpy as jnp
import numpy as np
from jax import lax
from jax.experimental import pallas as pl
from jax.experimental.pallas import tpu as pltpu
from jax.experimental.pallas import tpu_sc as plsc

_C = 128
_H = 128
_W = 128
_L = 8192
_S = 32
_F = 8
_NC = 2
_NS = 16
_PPT = 4
_PLANE = _H * _W
_LH = _L // _NC
_CHUNK = 256
_NPAIR = _LH // (2 * _CHUNK)
_GPC = _CHUNK // 16

_TVALS = [float(t) for t in np.linspace(0.0, 1.0, _S).astype(np.float32)]
_STEP = float(np.float32(1.0 / (_S - 1)))


def _body(feat_hbm, lines_hbm, out_hbm, feat_v, lines_v, stages, sems):
    lh = lax.axis_index("c")
    pp = lax.axis_index("s")

    pltpu.sync_copy(feat_hbm.at[pl.ds(pp * (_PPT * _PLANE), _PPT * _PLANE)],
                    feat_v)
    pltpu.sync_copy(lines_hbm.at[pl.ds(lh * (_LH * 4), _LH * 4)], lines_v)

    lane = lax.iota(jnp.int32, 16)
    planes = [feat_v.at[pl.ds(c * _PLANE, _PLANE)] for c in range(_PPT)]
    cols = [jnp.full((16,), k, jnp.int32) for k in range(_PPT * _F)]
    interleave = plsc.PackFormat.INTERLEAVED

    def make_group_body(stage_lo, stage_hi):
        def group_body(gi, lloc0):
            lloc = lloc0 + gi * 16
            li4 = (lloc + lane) * 4
            x1 = plsc.load_gather(lines_v, [li4])
            y1 = plsc.load_gather(lines_v, [li4 + 1])
            x2 = plsc.load_gather(lines_v, [li4 + 2])
            y2 = plsc.load_gather(lines_v, [li4 + 3])
            sx = (x1 - x2) * _STEP
            sy = (y1 - y2) * _STEP
            px = x2 - 0.5
            py = y2 - 0.5
            rows = gi * 16 + lane
            for f in range(_F):
                acc = [None] * _PPT
                for j in range(4):
                    x0i = px.astype(jnp.int32)
                    y0i = py.astype(jnp.int32)
                    x1i = jnp.minimum(x0i + 1, _W - 1)
                    y1i = jnp.minimum(y0i + 1, _H - 1)
                    wb = px - x0i.astype(jnp.float32)
                    wa = x1i.astype(jnp.float32) - px
                    wd = py - y0i.astype(jnp.float32)
                    wc = y1i.astype(jnp.float32) - py
                    w00 = plsc.pack(wa * wc, wa * wc, format=interleave)
                    w01 = plsc.pack(wb * wc, wb * wc, format=interleave)
                    w10 = plsc.pack(wa * wd, wa * wd, format=interleave)
                    w11 = plsc.pack(wb * wd, wb * wd, format=interleave)
                    iy0 = y0i * _W
                    iy1 = y1i * _W
                    i00 = iy0 + x0i
                    i01 = iy0 + x1i
                    i10 = iy1 + x0i
                    i11 = iy1 + x1i
                    for c in range(_PPT):
                        g00 = plsc.bitcast(
                            plsc.load_gather(planes[c], [i00]), jnp.bfloat16)
                        g01 = plsc.bitcast(
                            plsc.load_gather(planes[c], [i01]), jnp.bfloat16)
                        g10 = plsc.bitcast(
                            plsc.load_gather(planes[c], [i10]), jnp.bfloat16)
                        g11 = plsc.bitcast(
                            plsc.load_gather(planes[c], [i11]), jnp.bfloat16)
                        v = w00 * g00 + w01 * g01 + w10 * g10 + w11 * g11
                        acc[c] = v if j == 0 else jnp.maximum(acc[c], v)
                    px = px + sx
                    py = py + sy
                for c in range(_PPT):
                    vlo, vhi = plsc.unpack(acc[c], format=interleave,
                                           preferred_element_type=jnp.float32)
                    plsc.store_scatter(stage_lo, [rows, cols[c * _F + f]], vlo)
                    plsc.store_scatter(stage_hi, [rows, cols[c * _F + f]], vhi)
            return lloc0

        return group_body

    def out_slices(ci):
        rowbase = lh * _LH + ci * _CHUNK
        return (out_hbm.at[pl.ds(rowbase, _CHUNK), pl.ds(pp * 32, 32)],
                out_hbm.at[pl.ds(rowbase, _CHUNK), pl.ds(512 + pp * 32, 32)])

    def pair_body(cp, carry):
        for b in range(2):
            ci = cp * 2 + b
            stage_lo, stage_hi = stages[2 * b], stages[2 * b + 1]
            dst_lo, dst_hi = out_slices(ci)

            @pl.when(cp > 0)
            def _():
                pltpu.make_async_copy(stage_lo, dst_lo, sems[b]).wait()
                pltpu.make_async_copy(stage_hi, dst_hi, sems[b]).wait()

            lax.fori_loop(0, _GPC, make_group_body(stage_lo, stage_hi),
                          ci * _CHUNK)
            pltpu.async_copy(stage_lo, dst_lo, sems[b])
            pltpu.async_copy(stage_hi, dst_hi, sems[b])
        return carry

    lax.fori_loop(0, _NPAIR, pair_body, 0)
    for b in range(2):
        dst_lo, dst_hi = out_slices(2 * (_NPAIR - 1) + b)
        pltpu.make_async_copy(stages[2 * b], dst_lo, sems[b]).wait()
        pltpu.make_async_copy(stages[2 * b + 1], dst_hi, sems[b]).wait()


_sc_call = functools.partial(
    pl.kernel,
    out_type=jax.ShapeDtypeStruct((_L, _C * _F), jnp.float32),
    mesh=plsc.VectorSubcoreMesh(core_axis_name="c", subcore_axis_name="s"),
    compiler_params=pltpu.CompilerParams(use_tc_tiling_on_sc=False,
                                         needs_layout_passes=False),
    scratch_types=[
        pltpu.VMEM((_PPT * _PLANE,), jnp.int32),
        pltpu.VMEM((_LH * 4,), jnp.float32),
        [pltpu.VMEM((_CHUNK, 32), jnp.float32)] * 4,
        [pltpu.SemaphoreType.DMA] * 2,
    ],
)(_body)


def kernel(features_per_image, lines_per_im):
    fb = features_per_image.astype(jnp.bfloat16)
    lo = lax.bitcast_convert_type(fb[:64], jnp.uint16).astype(jnp.uint32)
    hi = lax.bitcast_convert_type(fb[64:], jnp.uint16).astype(jnp.uint32)
    packed = lax.bitcast_convert_type((hi << 16) | lo, jnp.int32).reshape(-1)
    lines_flat = lines_per_im.reshape(-1)
    return _sc_call(packed, lines_flat)
